# baseline (speedup 1.0000x reference)
import functools

import jax
import jax.numpy as jnp
from jax import lax
from jax.experimental import pallas as pl
from jax.experimental.pallas import tpu as pltpu
from jax.experimental.pallas import tpu_sc as plsc

N = 10000
D = 128
LANES = 16

NC = 2
NS = 16
NW = NC * NS

CHUNK = 64
CHUNKS_PER_WORKER = 160
SB = 8
NBLK = CHUNKS_PER_WORKER // SB
NBUF = 4
E_PAD = NW * CHUNKS_PER_WORKER * CHUNK
N_PAD = 10240
ROWS_PER_SUBCORE = N_PAD // NS


def _sc_aggregate(x_hbm, src_hbm, dst_hbm, a_hbm, zeros_hbm, out_hbm,
                  src_v, dst_v, a_v, rows_v, acc_sh, sem_i, sem_g, sem_s):
    c_ax = lax.axis_index("c")
    s_ax = lax.axis_index("s")
    wid = s_ax * NC + c_ax
    base = wid * CHUNKS_PER_WORKER

    pltpu.sync_copy(
        zeros_hbm,
        acc_sh.at[pl.ds(s_ax * ROWS_PER_SUBCORE, ROWS_PER_SUBCORE)])

    idx_pairs = ((src_hbm, src_v), (dst_hbm, dst_v), (a_hbm, a_v))

    def fire_idx(blk, ib):
        for ref_h, ref_v in idx_pairs:
            pltpu.async_copy(
                ref_h.at[pl.ds(base + blk * SB, SB)], ref_v.at[ib], sem_i)

    def wait_idx(blk, ib):
        for ref_h, ref_v in idx_pairs:
            pltpu.make_async_copy(
                ref_h.at[pl.ds(base + blk * SB, SB)], ref_v.at[ib],
                sem_i).wait()

    def fire_gather(ib, r, gb):
        pltpu.async_copy(x_hbm.at[src_v.at[ib, r]], rows_v.at[gb], sem_g)

    def wait_gather(ib, r, gb):
        pltpu.make_async_copy(
            x_hbm.at[src_v.at[ib, r]], rows_v.at[gb], sem_g).wait()

    def fire_scatter(ib, r, gb):
        pltpu.async_copy(
            rows_v.at[gb], acc_sh.at[dst_v.at[ib, r]], sem_s, add=True)

    def wait_scatter(ib, r, gb):
        pltpu.make_async_copy(
            rows_v.at[gb], acc_sh.at[dst_v.at[ib, r]], sem_s).wait()

    fire_idx(0, 0)
    wait_idx(0, 0)
    fire_idx(1, 1)
    fire_gather(0, 0, 0)
    fire_gather(0, 1, 1)
    fire_gather(0, 2, 2)
    plsc.subcore_barrier()

    def pipe_body(bi2, carry):
        for bb in range(2):
            for b in range(SB):
                ch = bi2 * (2 * SB) + bb * SB + b
                rb = b % NBUF

                wait_gather(bb, b, rb)

                p_ib, p_r = (bb, b - 1) if b >= 1 else (1 - bb, SB - 1)

                @pl.when(ch >= 1)
                def _():
                    wait_scatter(p_ib, p_r, (rb + 3) % NBUF)

                n_ib, n_r = (bb, b + 3) if b < SB - 3 else (1 - bb, b - 5)
                if b == SB - 3:
                    @pl.when(ch + 3 < CHUNKS_PER_WORKER)
                    def _():
                        wait_idx(bi2 * 2 + bb + 1, 1 - bb)
                        fire_gather(n_ib, n_r, (rb + 3) % NBUF)
                elif b > SB - 3:
                    @pl.when(ch + 3 < CHUNKS_PER_WORKER)
                    def _():
                        fire_gather(n_ib, n_r, (rb + 3) % NBUF)
                else:
                    fire_gather(n_ib, n_r, (rb + 3) % NBUF)

                if b == 1:
                    blk = bi2 * 2 + bb

                    @pl.when((ch >= SB) & (ch < (NBLK - 1) * SB + 1))
                    def _():
                        fire_idx(blk + 1, 1 - bb)

                def t_body(t, carry2):
                    av16 = a_v[bb, b, pl.ds(t * LANES, LANES)]
                    for k in range(LANES):
                        e = t * LANES + k
                        av = jnp.full((LANES,), av16[k], jnp.float32)
                        for j2 in range(D // LANES):
                            sl = (rb, e, pl.ds(j2 * LANES, LANES))
                            rows_v[sl] = rows_v[sl] * av
                    return carry2
                lax.fori_loop(0, CHUNK // LANES, t_body, 0)

                fire_scatter(bb, b, rb)
        return carry

    lax.fori_loop(0, NBLK // 2, pipe_body, 0)
    wait_scatter(1, SB - 1, (SB - 1) % NBUF)
    plsc.subcore_barrier()

    pltpu.sync_copy(
        acc_sh.at[pl.ds(s_ax * ROWS_PER_SUBCORE, ROWS_PER_SUBCORE)],
        out_hbm.at[c_ax, pl.ds(s_ax * ROWS_PER_SUBCORE, ROWS_PER_SUBCORE)])


_sc_agg_call = functools.partial(
    pl.kernel,
    out_type=jax.ShapeDtypeStruct((NC, N_PAD, D), jnp.float32),
    mesh=plsc.VectorSubcoreMesh(core_axis_name="c", subcore_axis_name="s"),
    scratch_types=[
        pltpu.VMEM((2, SB, CHUNK), jnp.int32),
        pltpu.VMEM((2, SB, CHUNK), jnp.int32),
        pltpu.VMEM((2, SB, CHUNK), jnp.float32),
        pltpu.VMEM((NBUF, CHUNK, D), jnp.float32),
        pltpu.VMEM_SHARED((N_PAD, D), jnp.float32),
        pltpu.SemaphoreType.DMA,
        pltpu.SemaphoreType.DMA,
        pltpu.SemaphoreType.DMA,
    ],
)(_sc_aggregate)


def _tc_finish(p_ref, w_ref, o_ref):
    h = p_ref[0] + p_ref[1]
    o_ref[...] = jnp.maximum(
        jnp.dot(h, w_ref[...], preferred_element_type=jnp.float32), 0.0)


@jax.jit
def kernel(X, edge_index, A_vals, W):
    e = edge_index.shape[1]
    n_pad = E_PAD - e
    pad_idx = jnp.arange(n_pad, dtype=jnp.int32) % N
    src_p = jnp.concatenate([edge_index[0], pad_idx]).reshape(-1, CHUNK)
    dst_p = jnp.concatenate([edge_index[1], pad_idx]).reshape(-1, CHUNK)
    a_p = jnp.concatenate(
        [A_vals, jnp.zeros((n_pad,), jnp.float32)]).reshape(-1, CHUNK)
    zeros = jnp.zeros((ROWS_PER_SUBCORE, D), jnp.float32)

    partials = _sc_agg_call(X, src_p, dst_p, a_p, zeros)

    rows_blk = 1000
    out = pl.pallas_call(
        _tc_finish,
        grid=(N // rows_blk,),
        in_specs=[
            pl.BlockSpec((NC, rows_blk, D), lambda i: (0, i, 0)),
            pl.BlockSpec((D, D), lambda i: (0, 0)),
        ],
        out_specs=pl.BlockSpec((rows_blk, D), lambda i: (i, 0)),
        out_shape=jax.ShapeDtypeStruct((N, D), jnp.float32),
    )(partials, W)
    return out


# speedup vs baseline: 1.0418x; 1.0418x over previous
import functools

import jax
import jax.numpy as jnp
from jax import lax
from jax.experimental import pallas as pl
from jax.experimental.pallas import tpu as pltpu
from jax.experimental.pallas import tpu_sc as plsc

N = 10000
D = 128
LANES = 16

NC = 2
NS = 16
NW = NC * NS

CHUNK = 80
CHUNKS_PER_WORKER = 128
SB = 8
NBLK = CHUNKS_PER_WORKER // SB
NBUF = 4
E_PAD = NW * CHUNKS_PER_WORKER * CHUNK
N_PAD = 10240
ROWS_PER_SUBCORE = N_PAD // NS


def _sc_aggregate(x_hbm, src_hbm, dst_hbm, a_hbm, zeros_hbm, out_hbm,
                  src_v, dst_v, a_v, rows_v, acc_sh, sem_i, sem_g, sem_s):
    c_ax = lax.axis_index("c")
    s_ax = lax.axis_index("s")
    wid = s_ax * NC + c_ax
    base = wid * CHUNKS_PER_WORKER

    pltpu.sync_copy(
        zeros_hbm,
        acc_sh.at[pl.ds(s_ax * ROWS_PER_SUBCORE, ROWS_PER_SUBCORE)])

    idx_pairs = ((src_hbm, src_v), (dst_hbm, dst_v), (a_hbm, a_v))

    def fire_idx(blk, ib):
        for ref_h, ref_v in idx_pairs:
            pltpu.async_copy(
                ref_h.at[pl.ds(base + blk * SB, SB)], ref_v.at[ib], sem_i)

    def wait_idx(blk, ib):
        for ref_h, ref_v in idx_pairs:
            pltpu.make_async_copy(
                ref_h.at[pl.ds(base + blk * SB, SB)], ref_v.at[ib],
                sem_i).wait()

    def fire_gather(ib, r, gb):
        pltpu.async_copy(x_hbm.at[src_v.at[ib, r]], rows_v.at[gb], sem_g)

    def wait_gather(ib, r, gb):
        pltpu.make_async_copy(
            x_hbm.at[src_v.at[ib, r]], rows_v.at[gb], sem_g).wait()

    def fire_scatter(ib, r, gb):
        pltpu.async_copy(
            rows_v.at[gb], acc_sh.at[dst_v.at[ib, r]], sem_s, add=True)

    def wait_scatter(ib, r, gb):
        pltpu.make_async_copy(
            rows_v.at[gb], acc_sh.at[dst_v.at[ib, r]], sem_s).wait()

    fire_idx(0, 0)
    wait_idx(0, 0)
    fire_idx(1, 1)
    fire_gather(0, 0, 0)
    fire_gather(0, 1, 1)
    plsc.subcore_barrier()

    def pipe_body(bi2, carry):
        for bb in range(2):
            for b in range(SB):
                ch = bi2 * (2 * SB) + bb * SB + b
                rb = b % NBUF

                wait_gather(bb, b, rb)

                p_ib, p_r = (bb, b - 2) if b >= 2 else (1 - bb, SB - 2 + b)

                @pl.when(ch >= 2)
                def _():
                    wait_scatter(p_ib, p_r, (rb + 2) % NBUF)

                n_ib, n_r = (bb, b + 2) if b < SB - 2 else (1 - bb, b - 6)
                if b == SB - 2:
                    @pl.when(ch + 2 < CHUNKS_PER_WORKER)
                    def _():
                        wait_idx(bi2 * 2 + bb + 1, 1 - bb)
                        fire_gather(n_ib, n_r, (rb + 2) % NBUF)
                elif b == SB - 1:
                    @pl.when(ch + 2 < CHUNKS_PER_WORKER)
                    def _():
                        fire_gather(n_ib, n_r, (rb + 2) % NBUF)
                else:
                    fire_gather(n_ib, n_r, (rb + 2) % NBUF)

                if b == 1:
                    blk = bi2 * 2 + bb

                    @pl.when((ch >= SB) & (ch < (NBLK - 1) * SB + 1))
                    def _():
                        fire_idx(blk + 1, 1 - bb)

                def t_body(t, carry2):
                    av16 = a_v[bb, b, pl.ds(t * LANES, LANES)]
                    for k in range(LANES):
                        e = t * LANES + k
                        av = jnp.full((LANES,), av16[k], jnp.float32)
                        for j2 in range(D // LANES):
                            sl = (rb, e, pl.ds(j2 * LANES, LANES))
                            rows_v[sl] = rows_v[sl] * av
                    return carry2
                lax.fori_loop(0, CHUNK // LANES, t_body, 0)

                fire_scatter(bb, b, rb)
        return carry

    lax.fori_loop(0, NBLK // 2, pipe_body, 0)
    wait_scatter(1, SB - 2, (SB - 2) % NBUF)
    wait_scatter(1, SB - 1, (SB - 1) % NBUF)
    plsc.subcore_barrier()

    pltpu.sync_copy(
        acc_sh.at[pl.ds(s_ax * ROWS_PER_SUBCORE, ROWS_PER_SUBCORE)],
        out_hbm.at[c_ax, pl.ds(s_ax * ROWS_PER_SUBCORE, ROWS_PER_SUBCORE)])


_sc_agg_call = functools.partial(
    pl.kernel,
    out_type=jax.ShapeDtypeStruct((NC, N_PAD, D), jnp.float32),
    mesh=plsc.VectorSubcoreMesh(core_axis_name="c", subcore_axis_name="s"),
    scratch_types=[
        pltpu.VMEM((2, SB, CHUNK), jnp.int32),
        pltpu.VMEM((2, SB, CHUNK), jnp.int32),
        pltpu.VMEM((2, SB, CHUNK), jnp.float32),
        pltpu.VMEM((NBUF, CHUNK, D), jnp.float32),
        pltpu.VMEM_SHARED((N_PAD, D), jnp.float32),
        pltpu.SemaphoreType.DMA,
        pltpu.SemaphoreType.DMA,
        pltpu.SemaphoreType.DMA,
    ],
)(_sc_aggregate)


def _tc_finish(p_ref, w_ref, o_ref):
    h = p_ref[0] + p_ref[1]
    o_ref[...] = jnp.maximum(
        jnp.dot(h, w_ref[...], preferred_element_type=jnp.float32), 0.0)


@jax.jit
def kernel(X, edge_index, A_vals, W):
    e = edge_index.shape[1]
    n_pad = E_PAD - e
    pad_idx = jnp.arange(n_pad, dtype=jnp.int32) % N
    src_p = jnp.concatenate([edge_index[0], pad_idx]).reshape(-1, CHUNK)
    dst_p = jnp.concatenate([edge_index[1], pad_idx]).reshape(-1, CHUNK)
    a_p = jnp.concatenate(
        [A_vals, jnp.zeros((n_pad,), jnp.float32)]).reshape(-1, CHUNK)
    zeros = jnp.zeros((ROWS_PER_SUBCORE, D), jnp.float32)

    partials = _sc_agg_call(X, src_p, dst_p, a_p, zeros)

    rows_blk = 1000
    out = pl.pallas_call(
        _tc_finish,
        grid=(N // rows_blk,),
        in_specs=[
            pl.BlockSpec((NC, rows_blk, D), lambda i: (0, i, 0)),
            pl.BlockSpec((D, D), lambda i: (0, 0)),
        ],
        out_specs=pl.BlockSpec((rows_blk, D), lambda i: (i, 0)),
        out_shape=jax.ShapeDtypeStruct((N, D), jnp.float32),
    )(partials, W)
    return out


# speedup vs baseline: 1.0531x; 1.0109x over previous
import functools

import numpy as np

import jax
import jax.numpy as jnp
from jax import lax
from jax.experimental import pallas as pl
from jax.experimental.pallas import tpu as pltpu
from jax.experimental.pallas import tpu_sc as plsc

N = 10000
D = 128
LANES = 16

NC = 2
NS = 16
NW = NC * NS

CHUNK = 80
CHUNKS_PER_WORKER = 128
SB = 8
NBLK = CHUNKS_PER_WORKER // SB
NBUF = 4
E_PAD = NW * CHUNKS_PER_WORKER * CHUNK
N_PAD = 10240
ROWS_PER_SUBCORE = N_PAD // NS


def _sc_aggregate(x_hbm, src_hbm, dst_hbm, a_hbm, zeros_hbm, out_hbm,
                  src_v, dst_v, a_v, rows_v, acc_sh, sem_i, sem_g, sem_s):
    c_ax = lax.axis_index("c")
    s_ax = lax.axis_index("s")
    wid = s_ax * NC + c_ax
    base = wid * CHUNKS_PER_WORKER

    pltpu.sync_copy(
        zeros_hbm,
        acc_sh.at[pl.ds(s_ax * ROWS_PER_SUBCORE, ROWS_PER_SUBCORE)])

    idx_pairs = ((src_hbm, src_v), (dst_hbm, dst_v), (a_hbm, a_v))

    def fire_idx(blk, ib):
        for ref_h, ref_v in idx_pairs:
            pltpu.async_copy(
                ref_h.at[pl.ds(base + blk * SB, SB)], ref_v.at[ib], sem_i)

    def wait_idx(blk, ib):
        for ref_h, ref_v in idx_pairs:
            pltpu.make_async_copy(
                ref_h.at[pl.ds(base + blk * SB, SB)], ref_v.at[ib],
                sem_i).wait()

    def fire_gather(ib, r, gb):
        pltpu.async_copy(x_hbm.at[src_v.at[ib, r]], rows_v.at[gb], sem_g)

    def wait_gather(ib, r, gb):
        pltpu.make_async_copy(
            x_hbm.at[src_v.at[ib, r]], rows_v.at[gb], sem_g).wait()

    def fire_scatter(ib, r, gb):
        pltpu.async_copy(
            rows_v.at[gb], acc_sh.at[dst_v.at[ib, r]], sem_s, add=True)

    def wait_scatter(ib, r, gb):
        pltpu.make_async_copy(
            rows_v.at[gb], acc_sh.at[dst_v.at[ib, r]], sem_s).wait()

    fire_idx(0, 0)
    wait_idx(0, 0)
    fire_idx(1, 1)
    fire_gather(0, 0, 0)
    fire_gather(0, 1, 1)
    plsc.subcore_barrier()

    def pipe_body(bi2, carry):
        for bb in range(2):
            for b in range(SB):
                ch = bi2 * (2 * SB) + bb * SB + b
                rb = b % NBUF

                wait_gather(bb, b, rb)

                p_ib, p_r = (bb, b - 2) if b >= 2 else (1 - bb, SB - 2 + b)

                @pl.when(ch >= 2)
                def _():
                    wait_scatter(p_ib, p_r, (rb + 2) % NBUF)

                n_ib, n_r = (bb, b + 2) if b < SB - 2 else (1 - bb, b - 6)
                if b == SB - 2:
                    @pl.when(ch + 2 < CHUNKS_PER_WORKER)
                    def _():
                        wait_idx(bi2 * 2 + bb + 1, 1 - bb)
                        fire_gather(n_ib, n_r, (rb + 2) % NBUF)
                elif b == SB - 1:
                    @pl.when(ch + 2 < CHUNKS_PER_WORKER)
                    def _():
                        fire_gather(n_ib, n_r, (rb + 2) % NBUF)
                else:
                    fire_gather(n_ib, n_r, (rb + 2) % NBUF)

                if b == 1:
                    blk = bi2 * 2 + bb

                    @pl.when((ch >= SB) & (ch < (NBLK - 1) * SB + 1))
                    def _():
                        fire_idx(blk + 1, 1 - bb)

                def t_body(t, carry2):
                    av16 = a_v[bb, b, pl.ds(t * LANES, LANES)]
                    for k in range(LANES):
                        e = t * LANES + k
                        av = jnp.full((LANES,), av16[k], jnp.float32)
                        for j2 in range(D // LANES):
                            sl = (rb, e, pl.ds(j2 * LANES, LANES))
                            rows_v[sl] = rows_v[sl] * av
                    return carry2
                lax.fori_loop(0, CHUNK // LANES, t_body, 0)

                fire_scatter(bb, b, rb)
        return carry

    lax.fori_loop(0, NBLK // 2, pipe_body, 0)
    wait_scatter(1, SB - 2, (SB - 2) % NBUF)
    wait_scatter(1, SB - 1, (SB - 1) % NBUF)
    plsc.subcore_barrier()

    pltpu.sync_copy(
        acc_sh.at[pl.ds(s_ax * ROWS_PER_SUBCORE, ROWS_PER_SUBCORE)],
        out_hbm.at[c_ax, pl.ds(s_ax * ROWS_PER_SUBCORE, ROWS_PER_SUBCORE)])


_sc_agg_call = functools.partial(
    pl.kernel,
    out_type=jax.ShapeDtypeStruct((NC, N_PAD, D), jnp.float32),
    mesh=plsc.VectorSubcoreMesh(core_axis_name="c", subcore_axis_name="s"),
    scratch_types=[
        pltpu.VMEM((2, SB, CHUNK), jnp.int32),
        pltpu.VMEM((2, SB, CHUNK), jnp.int32),
        pltpu.VMEM((2, SB, CHUNK), jnp.float32),
        pltpu.VMEM((NBUF, CHUNK, D), jnp.float32),
        pltpu.VMEM_SHARED((N_PAD, D), jnp.float32),
        pltpu.SemaphoreType.DMA,
        pltpu.SemaphoreType.DMA,
        pltpu.SemaphoreType.DMA,
    ],
)(_sc_aggregate)


def _tc_finish(p_ref, w_ref, o_ref):
    h = p_ref[0] + p_ref[1]
    o_ref[...] = jnp.maximum(
        jnp.dot(h, w_ref[...], preferred_element_type=jnp.float32), 0.0)


@jax.jit
def kernel(X, edge_index, A_vals, W):
    e = edge_index.shape[1]
    n_pad = E_PAD - e
    pad_idx = jnp.asarray(np.arange(n_pad, dtype=np.int32) % N)
    src_p = jnp.concatenate([edge_index[0], pad_idx]).reshape(-1, CHUNK)
    dst_p = jnp.concatenate([edge_index[1], pad_idx]).reshape(-1, CHUNK)
    a_p = jnp.concatenate(
        [A_vals, jnp.zeros((n_pad,), jnp.float32)]).reshape(-1, CHUNK)
    zeros = jnp.zeros((ROWS_PER_SUBCORE, D), jnp.float32)

    partials = _sc_agg_call(X, src_p, dst_p, a_p, zeros)

    rows_blk = 2000
    out = pl.pallas_call(
        _tc_finish,
        grid=(N // rows_blk,),
        in_specs=[
            pl.BlockSpec((NC, rows_blk, D), lambda i: (0, i, 0)),
            pl.BlockSpec((D, D), lambda i: (0, 0)),
        ],
        out_specs=pl.BlockSpec((rows_blk, D), lambda i: (i, 0)),
        out_shape=jax.ShapeDtypeStruct((N, D), jnp.float32),
    )(partials, W)
    return out
